# Initial kernel scaffold; baseline (speedup 1.0000x reference)
import jax
import jax.numpy as jnp
from jax.experimental import pallas as pl


def kernel(feat0, feat1, feat2, feat3, E0, E1, E2, E3, W, b):
    raise NotImplementedError("write your pallas kernel here")


# speedup vs baseline: 6.9393x; 6.9393x over previous
import functools

import jax
import jax.numpy as jnp
from jax import lax
from jax.experimental import pallas as pl
from jax.experimental.pallas import tpu as pltpu
from jax.experimental.pallas import tpu_sc as plsc

B, T = 1024, 200
N = B * T
DIMS = (32, 16, 16, 16)
OUT_DIM = 128

NC, NS = 2, 16
NW = NC * NS
ROWS_PER_W = N // NW
IDX_LANES = 128
IDX_ROWS_PER_W = ROWS_PER_W // IDX_LANES
CHUNK_IDX_ROWS = 10
CHUNK = CHUNK_IDX_ROWS * IDX_LANES
NCHUNK = IDX_ROWS_PER_W // CHUNK_IDX_ROWS


def _gather_body(f0, f1, f2, f3, e0, e1, e2, e3,
                 s0, s1, s2, s3, idx_v, rows32, rows16, sem):
    wid = lax.axis_index("s") * NC + lax.axis_index("c")
    base_r = wid * ROWS_PER_W

    def do_feature(f3d, tab, out, rows_v):
        pltpu.sync_copy(f3d.at[wid], idx_v)

        def chunk(c, carry):
            cps = [
                pltpu.make_async_copy(
                    tab.at[idx_v.at[c * CHUNK_IDX_ROWS + j]],
                    rows_v.at[pl.ds(j * IDX_LANES, IDX_LANES)],
                    sem,
                )
                for j in range(CHUNK_IDX_ROWS)
            ]
            for cp in cps:
                cp.start()
            for cp in cps:
                cp.wait()
            pltpu.sync_copy(rows_v, out.at[pl.ds(base_r + c * CHUNK, CHUNK)])
            return carry

        lax.fori_loop(0, NCHUNK, chunk, 0)

    do_feature(f0, e0, s0, rows32)
    do_feature(f1, e1, s1, rows16)
    do_feature(f2, e2, s2, rows16)
    do_feature(f3, e3, s3, rows16)


@jax.jit
def _sc_gather(f0, f1, f2, f3, e0, e1, e2, e3):
    mesh = plsc.VectorSubcoreMesh(core_axis_name="c", subcore_axis_name="s")
    return pl.kernel(
        _gather_body,
        out_type=[jax.ShapeDtypeStruct((N, d), jnp.float32) for d in DIMS],
        mesh=mesh,
        scratch_types=[
            pltpu.VMEM((IDX_ROWS_PER_W, IDX_LANES), jnp.int32),
            pltpu.VMEM((CHUNK, 32), jnp.float32),
            pltpu.VMEM((CHUNK, 16), jnp.float32),
            pltpu.SemaphoreType.DMA,
        ],
        compiler_params=pltpu.CompilerParams(use_tc_tiling_on_sc=False),
    )(f0, f1, f2, f3, e0, e1, e2, e3)


MM_BLK = 2048


def _mm_body(s0, s1, s2, s3, w0, w1, w2, w3, bias, o):
    acc = jnp.dot(s0[...], w0[...], preferred_element_type=jnp.float32)
    acc = acc + jnp.dot(s1[...], w1[...], preferred_element_type=jnp.float32)
    acc = acc + jnp.dot(s2[...], w2[...], preferred_element_type=jnp.float32)
    acc = acc + jnp.dot(s3[...], w3[...], preferred_element_type=jnp.float32)
    o[...] = acc + bias[0:1, :]


@jax.jit
def _tc_project(s0, s1, s2, s3, w0, w1, w2, w3, bias):
    grid = (N // MM_BLK,)
    in_specs = [
        pl.BlockSpec((MM_BLK, DIMS[i]), lambda i: (i, 0)) for i in range(4)
    ] + [
        pl.BlockSpec((DIMS[i], OUT_DIM), lambda i: (0, 0)) for i in range(4)
    ] + [pl.BlockSpec((8, OUT_DIM), lambda i: (0, 0))]
    return pl.pallas_call(
        _mm_body,
        grid=grid,
        in_specs=in_specs,
        out_specs=pl.BlockSpec((MM_BLK, OUT_DIM), lambda i: (i, 0)),
        out_shape=jax.ShapeDtypeStruct((N, OUT_DIM), jnp.float32),
    )(s0, s1, s2, s3, w0, w1, w2, w3, bias)


def kernel(feat0, feat1, feat2, feat3, E0, E1, E2, E3, W, b):
    fs = [f.reshape(NW, IDX_ROWS_PER_W, IDX_LANES)
          for f in (feat0, feat1, feat2, feat3)]
    s0, s1, s2, s3 = _sc_gather(*fs, E0, E1, E2, E3)
    w0 = W[0:32]
    w1 = W[32:48]
    w2 = W[48:64]
    w3 = W[64:80]
    bias = jnp.broadcast_to(b, (8, OUT_DIM))
    out = _tc_project(s0, s1, s2, s3, w0, w1, w2, w3, bias)
    return out.reshape(B, T, OUT_DIM)


# speedup vs baseline: 9.6623x; 1.3924x over previous
import functools

import jax
import jax.numpy as jnp
from jax import lax
from jax.experimental import pallas as pl
from jax.experimental.pallas import tpu as pltpu
from jax.experimental.pallas import tpu_sc as plsc

B, T = 1024, 200
N = B * T
DIMS = (32, 16, 16, 16)
OFFS = (0, 32, 48, 64)
PAD = 128
OUT_DIM = 128

NC, NS = 2, 16
NW = NC * NS
ROWS_PER_W = N // NW
IDX_LANES = 128
IDX_ROWS_PER_W = ROWS_PER_W // IDX_LANES
CHUNK_IDX_ROWS = 5
CHUNK = CHUNK_IDX_ROWS * IDX_LANES
NCHUNK = IDX_ROWS_PER_W // CHUNK_IDX_ROWS


def _gather_body(f0, f1, f2, f3, e0, e1, e2, e3, out,
                 i0, i1, i2, i3, r0, r1, r2, r3, zbuf, sem):
    wid = lax.axis_index("s") * NC + lax.axis_index("c")
    base_r = wid * ROWS_PER_W

    pltpu.sync_copy(f0.at[wid], i0)
    pltpu.sync_copy(f1.at[wid], i1)
    pltpu.sync_copy(f2.at[wid], i2)
    pltpu.sync_copy(f3.at[wid], i3)

    zero = jnp.zeros((16,), jnp.float32)

    def zrow(r, carry):
        zbuf[r, pl.ds(0, 16)] = zero
        zbuf[r, pl.ds(16, 16)] = zero
        zbuf[r, pl.ds(32, 16)] = zero
        return carry

    lax.fori_loop(0, CHUNK, zrow, 0)

    tabs = (e0, e1, e2, e3)
    idxs = (i0, i1, i2, i3)
    rbufs = (r0, r1, r2, r3)

    def chunk(c, carry):
        cps = []
        for t in range(4):
            for j in range(CHUNK_IDX_ROWS):
                cps.append(pltpu.make_async_copy(
                    tabs[t].at[idxs[t].at[c * CHUNK_IDX_ROWS + j]],
                    rbufs[t].at[pl.ds(j * IDX_LANES, IDX_LANES)],
                    sem,
                ))
        for cp in cps:
            cp.start()
        for cp in cps:
            cp.wait()
        rows = pl.ds(base_r + c * CHUNK, CHUNK)
        for t in range(4):
            pltpu.sync_copy(rbufs[t], out.at[rows, pl.ds(OFFS[t], DIMS[t])])
        pltpu.sync_copy(zbuf, out.at[rows, pl.ds(80, 48)])
        return carry

    lax.fori_loop(0, NCHUNK, chunk, 0)


@jax.jit
def _sc_gather(f0, f1, f2, f3, e0, e1, e2, e3):
    mesh = plsc.VectorSubcoreMesh(core_axis_name="c", subcore_axis_name="s")
    return pl.kernel(
        _gather_body,
        out_type=jax.ShapeDtypeStruct((N, PAD), jnp.float32),
        mesh=mesh,
        scratch_types=[
            pltpu.VMEM((IDX_ROWS_PER_W, IDX_LANES), jnp.int32),
            pltpu.VMEM((IDX_ROWS_PER_W, IDX_LANES), jnp.int32),
            pltpu.VMEM((IDX_ROWS_PER_W, IDX_LANES), jnp.int32),
            pltpu.VMEM((IDX_ROWS_PER_W, IDX_LANES), jnp.int32),
            pltpu.VMEM((CHUNK, 32), jnp.float32),
            pltpu.VMEM((CHUNK, 16), jnp.float32),
            pltpu.VMEM((CHUNK, 16), jnp.float32),
            pltpu.VMEM((CHUNK, 16), jnp.float32),
            pltpu.VMEM((CHUNK, 48), jnp.float32),
            pltpu.SemaphoreType.DMA,
        ],
        compiler_params=pltpu.CompilerParams(use_tc_tiling_on_sc=False),
    )(f0, f1, f2, f3, e0, e1, e2, e3)


MM_BLK = 2048


def _mm_body(s, w, bias, o):
    o[...] = jnp.dot(s[...], w[...],
                     preferred_element_type=jnp.float32) + bias[0:1, :]


@jax.jit
def _tc_project(s, w, bias):
    return pl.pallas_call(
        _mm_body,
        grid=(N // MM_BLK,),
        in_specs=[
            pl.BlockSpec((MM_BLK, PAD), lambda i: (i, 0)),
            pl.BlockSpec((PAD, OUT_DIM), lambda i: (0, 0)),
            pl.BlockSpec((8, OUT_DIM), lambda i: (0, 0)),
        ],
        out_specs=pl.BlockSpec((MM_BLK, OUT_DIM), lambda i: (i, 0)),
        out_shape=jax.ShapeDtypeStruct((N, OUT_DIM), jnp.float32),
    )(s, w, bias)


def kernel(feat0, feat1, feat2, feat3, E0, E1, E2, E3, W, b):
    fs = [f.reshape(NW, IDX_ROWS_PER_W, IDX_LANES)
          for f in (feat0, feat1, feat2, feat3)]
    s = _sc_gather(*fs, E0, E1, E2, E3)
    wp = jnp.zeros((PAD, OUT_DIM), jnp.float32).at[0:80, :].set(W)
    bias = jnp.broadcast_to(b, (8, OUT_DIM))
    out = _tc_project(s, wp, bias)
    return out.reshape(B, T, OUT_DIM)
